# baseline (speedup 1.0000x reference)
import functools

import jax
import jax.numpy as jnp
from jax import lax
from jax.experimental import pallas as pl
from jax.experimental.pallas import tpu as pltpu
from jax.experimental.pallas import tpu_sc as plsc


def kernel(indices, table):
    B, S = indices.shape
    V, D = table.shape
    N = B * S

    info = plsc.get_sparse_core_info()
    NC, NS = info.num_cores, info.num_subcores
    NW = NC * NS
    CHUNK = 128
    NBUF = 4
    NSP = 2
    assert N % (NW * CHUNK) == 0
    n_chunks = N // (NW * CHUNK)
    assert n_chunks % NBUF == 0 and n_chunks >= 3 * NBUF

    idx3 = indices.reshape(NW, n_chunks, CHUNK)

    mesh = plsc.VectorSubcoreMesh(core_axis_name="c", subcore_axis_name="s")

    @functools.partial(
        pl.kernel,
        mesh=mesh,
        out_type=jax.ShapeDtypeStruct((N, D), jnp.float32),
        scratch_types=(
            [pltpu.VMEM((n_chunks, CHUNK), jnp.int32)]
            + [pltpu.VMEM((CHUNK, D), jnp.float32)] * NBUF
            + [pltpu.VMEM_SHARED((NS, NSP, CHUNK, D), jnp.float32)]
            + [pltpu.SemaphoreType.DMA] * (NBUF + 2 * NSP)
        ),
    )
    def sc_gather(idx_hbm, table_hbm, out_hbm, idx_v, *rest):
        rows = rest[:NBUF]
        sp = rest[NBUF]
        gsem = rest[NBUF + 1:2 * NBUF + 1]
        csem = rest[2 * NBUF + 1:2 * NBUF + 1 + NSP]
        wsem = rest[2 * NBUF + 1 + NSP:]
        sid = lax.axis_index("s")
        wid = sid * NC + lax.axis_index("c")
        base = wid * (n_chunks * CHUNK)
        pltpu.sync_copy(idx_hbm.at[wid], idx_v)

        def gather(j, b):
            return pltpu.make_async_copy(
                table_hbm.at[idx_v.at[j]], rows[b], gsem[b])

        def xbar(b, p):
            return pltpu.make_async_copy(rows[b], sp.at[sid, p], csem[p])

        def write(j, p):
            return pltpu.make_async_copy(
                sp.at[sid, p], out_hbm.at[pl.ds(base + j * CHUNK, CHUNK)],
                wsem[p])

        def step(j, b, p, issue_gather=True):
            bm, pm = (b - 1) % NBUF, (p - 1) % NSP
            gather(j, b).wait()
            write(j - NSP, p).wait()
            xbar(b, p).start()
            xbar(bm, pm).wait()
            write(j - 1, pm).start()
            if issue_gather:
                gather(j + NBUF - 1, bm).start()

        for j in range(NBUF):
            gather(j, j).start()
        gather(0, 0).wait()
        xbar(0, 0).start()
        for j in range(1, NBUF):
            b, p = j, j % NSP
            bm, pm = b - 1, (p - 1) % NSP
            gather(j, b).wait()
            if j >= NSP:
                write(j - NSP, p).wait()
            xbar(b, p).start()
            xbar(bm, pm).wait()
            write(j - 1, pm).start()
            gather(j + NBUF - 1, bm).start()

        def body(g, carry):
            j0 = NBUF * g
            for b in range(NBUF):
                step(j0 + b, b, b % NSP)
            return carry

        lax.fori_loop(1, n_chunks // NBUF - 1, body, 0)

        jt = n_chunks - NBUF
        step(jt, jt % NBUF, jt % NSP, issue_gather=True)
        for j in range(jt + 1, n_chunks):
            step(j, j % NBUF, j % NSP, issue_gather=False)
        jl = n_chunks - 1
        xbar(jl % NBUF, jl % NSP).wait()
        write(jl, jl % NSP).start()
        for j in range(n_chunks - NSP, n_chunks):
            write(j, j % NSP).wait()

    out = sc_gather(idx3, table)
    return out.reshape(B, S, D)


# speedup vs baseline: 1.0149x; 1.0149x over previous
import functools

import jax
import jax.numpy as jnp
from jax import lax
from jax.experimental import pallas as pl
from jax.experimental.pallas import tpu as pltpu
from jax.experimental.pallas import tpu_sc as plsc


def kernel(indices, table):
    B, S = indices.shape
    V, D = table.shape
    N = B * S

    info = plsc.get_sparse_core_info()
    NC, NS = info.num_cores, info.num_subcores
    NW = NC * NS
    CHUNK = 128
    NBUF = 4
    NSP = 2
    assert N % (NW * CHUNK) == 0
    n_chunks = N // (NW * CHUNK)
    assert n_chunks % NBUF == 0 and n_chunks >= 3 * NBUF

    idx3 = indices.reshape(NW, n_chunks, CHUNK)

    mesh = plsc.VectorSubcoreMesh(core_axis_name="c", subcore_axis_name="s")

    @functools.partial(
        pl.kernel,
        mesh=mesh,
        out_type=jax.ShapeDtypeStruct((N, D), jnp.float32),
        scratch_types=(
            [pltpu.VMEM((n_chunks, CHUNK), jnp.int32)]
            + [pltpu.VMEM((CHUNK, D), jnp.float32)] * NBUF
            + [pltpu.VMEM_SHARED((NS, NSP, CHUNK, D), jnp.float32)]
            + [pltpu.SemaphoreType.DMA] * (NBUF + 2 * NSP)
        ),
    )
    def sc_gather(idx_hbm, table_hbm, out_hbm, idx_v, *rest):
        rows = rest[:NBUF]
        sp = rest[NBUF]
        gsem = rest[NBUF + 1:2 * NBUF + 1]
        csem = rest[2 * NBUF + 1:2 * NBUF + 1 + NSP]
        wsem = rest[2 * NBUF + 1 + NSP:]
        sid = lax.axis_index("s")
        wid = sid * NC + lax.axis_index("c")
        base = wid * (n_chunks * CHUNK)
        pltpu.sync_copy(idx_hbm.at[wid], idx_v)

        def gather(j, b):
            return pltpu.make_async_copy(
                table_hbm.at[idx_v.at[j]], rows[b], gsem[b])

        def xbar(b, p):
            return pltpu.make_async_copy(rows[b], sp.at[sid, p], csem[p])

        def write(j, p):
            return pltpu.make_async_copy(
                sp.at[sid, p], out_hbm.at[pl.ds(base + j * CHUNK, CHUNK)],
                wsem[p])

        def step(j, b, p):
            gather(j, b).wait()
            write(j - NSP, p).wait()
            xbar(b, p).start()
            xbar(b, p).wait()
            write(j, p).start()
            gather(j + NBUF, b).start()

        for j in range(NBUF):
            gather(j, j).start()
        for j in range(NBUF):
            b, p = j, j % NSP
            gather(j, b).wait()
            if j >= NSP:
                write(j - NSP, p).wait()
            xbar(b, p).start()
            xbar(b, p).wait()
            write(j, p).start()
            gather(j + NBUF, b).start()

        def body(g, carry):
            j0 = NBUF * g
            for b in range(NBUF):
                step(j0 + b, b, b % NSP)
            return carry

        lax.fori_loop(1, n_chunks // NBUF - 1, body, 0)

        for j in range(n_chunks - NBUF, n_chunks):
            b, p = j % NBUF, j % NSP
            gather(j, b).wait()
            write(j - NSP, p).wait()
            xbar(b, p).start()
            xbar(b, p).wait()
            write(j, p).start()
        for j in range(n_chunks - NSP, n_chunks):
            write(j, j % NSP).wait()

    out = sc_gather(idx3, table)
    return out.reshape(B, S, D)
